# baseline (speedup 1.0000x reference)
import functools

import jax
import jax.numpy as jnp
from jax import lax
from jax.experimental import pallas as pl
from jax.experimental.pallas import tpu as pltpu
from jax.experimental.pallas import tpu_sc as plsc

N = 10000
D = 128
G = 64
NC = 2
NS = 16
NW = NC * NS
B = 128
NREAL_H = 5000
PADB = 1144
HALF = NREAL_H + PADB
NPAD = 2 * HALF
RPH = HALF // NS
RPT = NPAD // NS
E = 320000
NCHUNK = -(-E // (NW * B))
NCHUNK += NCHUNK % 2
EPAD = NW * B * NCHUNK
CCAP = NCHUNK + 2

_HIGH = lax.Precision.HIGHEST


def _sc_mesh():
    return plsc.VectorSubcoreMesh(core_axis_name="c", subcore_axis_name="s",
                                  num_cores=NC, num_subcores=NS)


@functools.cache
def _get_part_kernel():
    return functools.partial(
        pl.kernel,
        out_type=(
            jax.ShapeDtypeStruct((NC, NPAD), jnp.float32),
            jax.ShapeDtypeStruct((2 * NW, CCAP, B), jnp.int32),
            jax.ShapeDtypeStruct((2 * NW, CCAP, B), jnp.int32),
            jax.ShapeDtypeStruct((NW, 1, B), jnp.int32),
        ),
        mesh=_sc_mesh(),
        compiler_params=pltpu.CompilerParams(needs_layout_passes=False),
        scratch_types=[
            pltpu.VMEM((NCHUNK, B), jnp.int32),
            pltpu.VMEM((NCHUNK, B), jnp.int32),
            pltpu.VMEM((CCAP, B), jnp.int32),
            pltpu.VMEM((CCAP, B), jnp.int32),
            pltpu.VMEM((CCAP, B), jnp.int32),
            pltpu.VMEM((CCAP, B), jnp.int32),
            pltpu.VMEM((B,), jnp.float32),
            pltpu.VMEM((RPT,), jnp.float32),
            pltpu.VMEM((1, B), jnp.int32),
            pltpu.VMEM_SHARED((NPAD,), jnp.float32),
        ],
    )(_part_body)


def _part_body(srcw_hbm, dstw_hbm, deg_hbm, srcl_hbm, dstl_hbm, cnt_hbm,
               src_v, dst_v, s0_v, d0_v, s1_v, d1_v, ones_v, zero_v, cnt_v,
               deg_sp):
    c = lax.axis_index("c")
    s = lax.axis_index("s")
    wid = c * NS + s
    t0 = s * RPT
    for k in range(B // 16):
        ones_v[pl.ds(k * 16, 16)] = jnp.ones((16,), jnp.float32)
    for k in range(RPT // 16):
        zero_v[pl.ds(k * 16, 16)] = jnp.zeros((16,), jnp.float32)
    pltpu.sync_copy(srcw_hbm.at[wid], src_v)
    pltpu.sync_copy(dstw_hbm.at[wid], dst_v)
    pltpu.sync_copy(zero_v, deg_sp.at[pl.ds(t0, RPT)])
    plsc.subcore_barrier()

    iota16 = lax.iota(jnp.int32, 16)

    def scat2(ref_s, ref_d, pos, vs, vd, m=None):
        pj = lax.shift_right_logical(pos, 7)
        pk = lax.bitwise_and(pos, 127)
        if m is None:
            plsc.store_scatter(ref_s, [pj, pk], vs)
            plsc.store_scatter(ref_d, [pj, pk], vd)
        else:
            plsc.store_scatter(ref_s, [pj, pk], vs, mask=m)
            plsc.store_scatter(ref_d, [pj, pk], vd, mask=m)

    def chunk(j, offs):
        off0, off1 = offs
        pltpu.sync_copy(ones_v, deg_sp.at[dst_v.at[j]], add=True)
        for k in range(B // 16):
            vs = src_v[j, pl.ds(k * 16, 16)]
            vd = dst_v[j, pl.ds(k * 16, 16)]
            m0 = vd < HALF
            cs0 = plsc.cumsum(m0.astype(jnp.int32))
            scat2(s0_v, d0_v, off0 + cs0 - 1, vs, vd, m0)
            off0 = off0 + jnp.max(cs0)
            m1 = jnp.logical_not(m0)
            cs1 = plsc.cumsum(m1.astype(jnp.int32))
            scat2(s1_v, d1_v, off1 + cs1 - 1, vs, vd - HALF, m1)
            off1 = off1 + jnp.max(cs1)
        return off0, off1

    off0, off1 = lax.fori_loop(0, NCHUNK, chunk, (jnp.int32(0), jnp.int32(0)))

    for k in range(2 * B // 16):
        ps = (iota16 * 607 + k * 131) % NREAL_H
        pd = NREAL_H + ((iota16 + k * 16) % PADB)
        scat2(s0_v, d0_v, off0 + k * 16 + iota16, ps, pd)
        scat2(s1_v, d1_v, off1 + k * 16 + iota16, ps, pd)
    ch0 = (off0 + B - 1) // B
    ch0 = ch0 + (ch0 & 1)
    ch1 = (off1 + B - 1) // B
    ch1 = ch1 + (ch1 & 1)
    cnt_v[0, pl.ds(0, 16)] = jnp.full((16,), ch0, jnp.int32)
    cnt_v[0, pl.ds(16, 16)] = jnp.full((16,), ch1, jnp.int32)

    pltpu.sync_copy(s0_v, srcl_hbm.at[2 * wid])
    pltpu.sync_copy(d0_v, dstl_hbm.at[2 * wid])
    pltpu.sync_copy(s1_v, srcl_hbm.at[2 * wid + 1])
    pltpu.sync_copy(d1_v, dstl_hbm.at[2 * wid + 1])
    pltpu.sync_copy(cnt_v, cnt_hbm.at[wid])
    plsc.subcore_barrier()
    pltpu.sync_copy(deg_sp.at[pl.ds(t0, RPT)], deg_hbm.at[c, pl.ds(t0, RPT)])


@functools.cache
def _get_edge_kernel():
    return functools.partial(
        pl.kernel,
        out_type=jax.ShapeDtypeStruct((NC, HALF, D), jnp.float32),
        mesh=_sc_mesh(),
        compiler_params=pltpu.CompilerParams(needs_layout_passes=False),
        scratch_types=[
            pltpu.VMEM((CCAP, B), jnp.int32),
            pltpu.VMEM((CCAP, B), jnp.int32),
            pltpu.VMEM((CCAP, B), jnp.int32),
            pltpu.VMEM((CCAP, B), jnp.int32),
            pltpu.VMEM((1, B), jnp.int32),
            pltpu.VMEM((1, B), jnp.int32),
            pltpu.VMEM((B, D), jnp.float32),
            pltpu.VMEM((B, D), jnp.float32),
            pltpu.VMEM_SHARED((HALF, D), jnp.float32),
            pltpu.SemaphoreType.DMA,
            pltpu.SemaphoreType.DMA,
            pltpu.SemaphoreType.DMA,
            pltpu.SemaphoreType.DMA,
        ],
    )(_edge_body)


def _edge_body(y_hbm, srcl_hbm, dstl_hbm, cnt_hbm, out_hbm,
               sidx0, sidx1, didx0, didx1, cv0, cv1, g0, g1, z_sp,
               sem0, sem1, sems0, sems1):
    c = lax.axis_index("c")
    s = lax.axis_index("s")
    t0 = s * RPH
    pltpu.sync_copy(y_hbm.at[pl.ds(c * HALF + t0, RPH), :],
                    z_sp.at[pl.ds(t0, RPH), :])
    pltpu.sync_copy(srcl_hbm.at[4 * s + c], sidx0)
    pltpu.sync_copy(srcl_hbm.at[4 * s + 2 + c], sidx1)
    pltpu.sync_copy(dstl_hbm.at[4 * s + c], didx0)
    pltpu.sync_copy(dstl_hbm.at[4 * s + 2 + c], didx1)
    pltpu.sync_copy(cnt_hbm.at[2 * s], cv0)
    pltpu.sync_copy(cnt_hbm.at[2 * s + 1], cv1)
    plsc.subcore_barrier()

    for sidx, didx, cvx in ((sidx0, didx0, cv0), (sidx1, didx1, cv1)):
        ct = jnp.max(cvx[0, pl.ds(c * 16, 16)])

        @pl.when(ct >= 2)
        def _(sidx=sidx):
            pltpu.async_copy(y_hbm.at[sidx.at[0]], g0, sem0)
            pltpu.async_copy(y_hbm.at[sidx.at[1]], g1, sem1)

        def pair(jj, carry, sidx=sidx, didx=didx, ct=ct):
            j = jj * 2
            pltpu.make_async_copy(y_hbm.at[sidx.at[0]], g0, sem0).wait()
            pltpu.async_copy(g0, z_sp.at[didx.at[j]], sems0, add=True)
            pltpu.make_async_copy(y_hbm.at[sidx.at[0]], g1, sem1).wait()
            pltpu.async_copy(g1, z_sp.at[didx.at[j + 1]], sems1, add=True)

            @pl.when(j + 2 < ct)
            def _():
                pltpu.make_async_copy(g0, z_sp.at[didx.at[0]], sems0).wait()
                pltpu.async_copy(y_hbm.at[sidx.at[j + 2]], g0, sem0)

            @pl.when(j + 3 < ct)
            def _():
                pltpu.make_async_copy(g1, z_sp.at[didx.at[0]], sems1).wait()
                pltpu.async_copy(y_hbm.at[sidx.at[j + 3]], g1, sem1)

            return carry

        lax.fori_loop(0, ct // 2, pair, 0)

        @pl.when(ct >= 2)
        def _(didx=didx):
            pltpu.make_async_copy(g0, z_sp.at[didx.at[0]], sems0).wait()
            pltpu.make_async_copy(g1, z_sp.at[didx.at[0]], sems1).wait()

    plsc.subcore_barrier()
    pltpu.sync_copy(z_sp.at[pl.ds(t0, RPH), :],
                    out_hbm.at[c, pl.ds(t0, RPH), :])


_BR = 256
_GRID = NPAD // _BR


def _row_spec():
    return pl.BlockSpec((_BR, D), lambda i: (i, 0))


def _vec_spec():
    return pl.BlockSpec((_BR,), lambda i: (i,))


def _full_spec(shape):
    nd = len(shape)
    return pl.BlockSpec(shape, lambda i: (0,) * nd)


def _mm0_body(x_ref, w_ref, da_ref, db_ref, y_ref):
    dinv = lax.rsqrt(da_ref[...] + db_ref[...] + 1.0)
    xw = lax.dot_general(x_ref[...], w_ref[...], (((1,), (0,)), ((), ())),
                         precision=_HIGH, preferred_element_type=jnp.float32)
    y_ref[...] = xw * dinv[:, None]


def _mm12_body(z_ref, da_ref, db_ref, b_ref, w_ref, y_ref):
    dinv = lax.rsqrt(da_ref[...] + db_ref[...] + 1.0)
    h = jnp.maximum(z_ref[...] * dinv[:, None] + b_ref[...][None, :], 0.0)
    hw = lax.dot_general(h, w_ref[...], (((1,), (0,)), ((), ())),
                         precision=_HIGH, preferred_element_type=jnp.float32)
    y_ref[...] = hw * dinv[:, None]


def _tail_body(z_ref, da_ref, db_ref, b_ref, wn_ref, bn_ref, batch_ref,
               w1_ref, b1_ref, w2_ref, b2_ref,
               np_ref, sums_ref, cnt_ref, fea_ref):
    i = pl.program_id(0)
    dinv = lax.rsqrt(da_ref[...] + db_ref[...] + 1.0)
    h = jnp.maximum(z_ref[...] * dinv[:, None] + b_ref[...][None, :], 0.0)
    np_ref[...] = jnp.sum(h * wn_ref[...][None, :], axis=1) + bn_ref[0]
    onehot = (batch_ref[...][:, None]
              == lax.broadcasted_iota(jnp.int32, (_BR, G), 1)).astype(jnp.float32)
    part = lax.dot_general(onehot, h, (((0,), (0,)), ((), ())),
                           precision=_HIGH, preferred_element_type=jnp.float32)
    cpart = jnp.sum(onehot, axis=0)

    @pl.when(i == 0)
    def _():
        sums_ref[...] = part
        cnt_ref[...] = cpart

    @pl.when(i > 0)
    def _():
        sums_ref[...] += part
        cnt_ref[...] += cpart

    @pl.when(i == _GRID - 1)
    def _():
        mean = sums_ref[...] / jnp.maximum(cnt_ref[...], 1.0)[:, None]
        t = lax.dot_general(mean, w1_ref[...], (((1,), (0,)), ((), ())),
                            precision=_HIGH, preferred_element_type=jnp.float32)
        t = jnp.maximum(t + b1_ref[...][None, :], 0.0)
        fea_ref[...] = lax.dot_general(t, w2_ref[...], (((1,), (0,)), ((), ())),
                                       precision=_HIGH,
                                       preferred_element_type=jnp.float32) \
            + b2_ref[...][None, :]


def _mm0(x_pad, w, dega, degb):
    return pl.pallas_call(
        _mm0_body,
        grid=(_GRID,),
        in_specs=[_row_spec(), _full_spec((D, D)), _vec_spec(), _vec_spec()],
        out_specs=_row_spec(),
        out_shape=jax.ShapeDtypeStruct((NPAD, D), jnp.float32),
    )(x_pad, w, dega, degb)


def _mm12(z, dega, degb, b, w):
    return pl.pallas_call(
        _mm12_body,
        grid=(_GRID,),
        in_specs=[_row_spec(), _vec_spec(), _vec_spec(),
                  _full_spec((D,)), _full_spec((D, D))],
        out_specs=_row_spec(),
        out_shape=jax.ShapeDtypeStruct((NPAD, D), jnp.float32),
    )(z, dega, degb, b, w)


def _tail(z, dega, degb, b, wn, bn, batch_pad, w1, b1, w2, b2):
    return pl.pallas_call(
        _tail_body,
        grid=(_GRID,),
        in_specs=[_row_spec(), _vec_spec(), _vec_spec(), _full_spec((D,)),
                  _full_spec((D,)), _full_spec((8,)), _vec_spec(),
                  _full_spec((D, 256)), _full_spec((256,)),
                  _full_spec((256, D)), _full_spec((D,))],
        out_specs=[_vec_spec(), _full_spec((G, D)), _full_spec((G,)),
                   _full_spec((G, D))],
        out_shape=[jax.ShapeDtypeStruct((NPAD,), jnp.float32),
                   jax.ShapeDtypeStruct((G, D), jnp.float32),
                   jax.ShapeDtypeStruct((G,), jnp.float32),
                   jax.ShapeDtypeStruct((G, D), jnp.float32)],
    )(z, dega, degb, b, wn, bn, batch_pad, w1, b1, w2, b2)


def kernel(x, edge_index, batch, W0, b0, W1, b1, W2, b2,
           W_f1, b_f1, W_f2, b_f2, W_n, b_n):
    zpad = jnp.zeros((PADB, D), jnp.float32)
    x_adj = jnp.concatenate([x[:NREAL_H], zpad, x[NREAL_H:], zpad])
    gfill = jnp.full((PADB,), G, jnp.int32)
    batch_adj = jnp.concatenate([batch[:NREAL_H], gfill, batch[NREAL_H:], gfill])
    src = edge_index[0]
    dst = edge_index[1]
    src_adj = jnp.where(src >= NREAL_H, src + PADB, src)
    dst_adj = jnp.where(dst >= NREAL_H, dst + PADB, dst)
    pad_i = jnp.arange(EPAD - E, dtype=jnp.int32)
    pad_src = (pad_i * 97) % NREAL_H
    pad_dst = NREAL_H + pad_i % PADB
    srcw = jnp.concatenate([src_adj, pad_src]).reshape(NW, NCHUNK, B)
    dstw = jnp.concatenate([dst_adj, pad_dst]).reshape(NW, NCHUNK, B)

    bn8 = jnp.pad(b_n, (0, 7))
    wn = W_n[:, 0]

    deg2, srcl, dstl, cnts = _get_part_kernel()(srcw, dstw)
    dega, degb = deg2[0], deg2[1]
    edge_k = _get_edge_kernel()

    y0 = _mm0(x_adj, W0, dega, degb)
    z0 = edge_k(y0, srcl, dstl, cnts).reshape(NPAD, D)
    y1 = _mm12(z0, dega, degb, b0, W1)
    z1 = edge_k(y1, srcl, dstl, cnts).reshape(NPAD, D)
    y2 = _mm12(z1, dega, degb, b1, W2)
    z2 = edge_k(y2, srcl, dstl, cnts).reshape(NPAD, D)

    node_prob, _, _, fea = _tail(z2, dega, degb, b2, wn, bn8, batch_adj,
                                 W_f1, b_f1, W_f2, b_f2)
    node_prob = jnp.concatenate([node_prob[:NREAL_H],
                                 node_prob[HALF:HALF + NREAL_H]])
    return (node_prob, fea)


# speedup vs baseline: 1.2694x; 1.2694x over previous
import functools

import jax
import jax.numpy as jnp
from jax import lax
from jax.experimental import pallas as pl
from jax.experimental.pallas import tpu as pltpu
from jax.experimental.pallas import tpu_sc as plsc

N = 10000
D = 128
G = 64
NC = 2
NS = 16
NW = NC * NS
B = 128
NREAL_H = 5000
PADB = 1144
HALF = NREAL_H + PADB
NPAD = 2 * HALF
RPH = HALF // NS
RPT = NPAD // NS
E = 320000
NCHUNK = -(-E // (NW * B))
NCHUNK += NCHUNK % 2
EPAD = NW * B * NCHUNK
CCAP = NCHUNK + 2

_HIGH = lax.Precision.HIGHEST


def _sc_mesh():
    return plsc.VectorSubcoreMesh(core_axis_name="c", subcore_axis_name="s",
                                  num_cores=NC, num_subcores=NS)


@functools.cache
def _get_part_kernel():
    return functools.partial(
        pl.kernel,
        out_type=(
            jax.ShapeDtypeStruct((NC, NPAD), jnp.float32),
            jax.ShapeDtypeStruct((2 * NW, CCAP, B), jnp.int32),
            jax.ShapeDtypeStruct((2 * NW, CCAP, B), jnp.int32),
            jax.ShapeDtypeStruct((NW, 1, B), jnp.int32),
        ),
        mesh=_sc_mesh(),
        compiler_params=pltpu.CompilerParams(needs_layout_passes=False),
        scratch_types=[
            pltpu.VMEM((NCHUNK, B), jnp.int32),
            pltpu.VMEM((NCHUNK, B), jnp.int32),
            pltpu.VMEM((CCAP, B), jnp.int32),
            pltpu.VMEM((CCAP, B), jnp.int32),
            pltpu.VMEM((CCAP, B), jnp.int32),
            pltpu.VMEM((CCAP, B), jnp.int32),
            pltpu.VMEM((B,), jnp.float32),
            pltpu.VMEM((RPT,), jnp.float32),
            pltpu.VMEM((1, B), jnp.int32),
            pltpu.VMEM_SHARED((NPAD,), jnp.float32),
        ],
    )(_part_body)


def _part_body(srcw_hbm, dstw_hbm, deg_hbm, srcl_hbm, dstl_hbm, cnt_hbm,
               src_v, dst_v, s0_v, d0_v, s1_v, d1_v, ones_v, zero_v, cnt_v,
               deg_sp):
    c = lax.axis_index("c")
    s = lax.axis_index("s")
    wid = c * NS + s
    t0 = s * RPT
    for k in range(B // 16):
        ones_v[pl.ds(k * 16, 16)] = jnp.ones((16,), jnp.float32)
    for k in range(RPT // 16):
        zero_v[pl.ds(k * 16, 16)] = jnp.zeros((16,), jnp.float32)
    pltpu.sync_copy(srcw_hbm.at[wid], src_v)
    pltpu.sync_copy(dstw_hbm.at[wid], dst_v)
    pltpu.sync_copy(zero_v, deg_sp.at[pl.ds(t0, RPT)])
    plsc.subcore_barrier()

    iota16 = lax.iota(jnp.int32, 16)

    def scat2(ref_s, ref_d, pos, vs, vd, m=None):
        pj = lax.shift_right_logical(pos, 7)
        pk = lax.bitwise_and(pos, 127)
        if m is None:
            plsc.store_scatter(ref_s, [pj, pk], vs)
            plsc.store_scatter(ref_d, [pj, pk], vd)
        else:
            plsc.store_scatter(ref_s, [pj, pk], vs, mask=m)
            plsc.store_scatter(ref_d, [pj, pk], vd, mask=m)

    def chunk(j, offs):
        off0, off1 = offs
        pltpu.sync_copy(ones_v, deg_sp.at[dst_v.at[j]], add=True)
        for k in range(B // 16):
            vs = src_v[j, pl.ds(k * 16, 16)]
            vd = dst_v[j, pl.ds(k * 16, 16)]
            m0 = vd < HALF
            cs0 = plsc.cumsum(m0.astype(jnp.int32))
            scat2(s0_v, d0_v, off0 + cs0 - 1, vs, vd, m0)
            off0 = off0 + jnp.max(cs0)
            m1 = jnp.logical_not(m0)
            cs1 = plsc.cumsum(m1.astype(jnp.int32))
            scat2(s1_v, d1_v, off1 + cs1 - 1, vs, vd - HALF, m1)
            off1 = off1 + jnp.max(cs1)
        return off0, off1

    off0, off1 = lax.fori_loop(0, NCHUNK, chunk, (jnp.int32(0), jnp.int32(0)))

    for k in range(2 * B // 16):
        ps = (iota16 * 607 + k * 131) % NREAL_H
        pd = NREAL_H + ((iota16 + k * 16) % PADB)
        scat2(s0_v, d0_v, off0 + k * 16 + iota16, ps, pd)
        scat2(s1_v, d1_v, off1 + k * 16 + iota16, ps, pd)
    ch0 = (off0 + B - 1) // B
    ch0 = ch0 + (ch0 & 1)
    ch1 = (off1 + B - 1) // B
    ch1 = ch1 + (ch1 & 1)
    cnt_v[0, pl.ds(0, 16)] = jnp.full((16,), ch0, jnp.int32)
    cnt_v[0, pl.ds(16, 16)] = jnp.full((16,), ch1, jnp.int32)

    pltpu.sync_copy(s0_v, srcl_hbm.at[2 * wid])
    pltpu.sync_copy(d0_v, dstl_hbm.at[2 * wid])
    pltpu.sync_copy(s1_v, srcl_hbm.at[2 * wid + 1])
    pltpu.sync_copy(d1_v, dstl_hbm.at[2 * wid + 1])
    pltpu.sync_copy(cnt_v, cnt_hbm.at[wid])
    plsc.subcore_barrier()
    pltpu.sync_copy(deg_sp.at[pl.ds(t0, RPT)], deg_hbm.at[c, pl.ds(t0, RPT)])


@functools.cache
def _get_edge_kernel():
    return functools.partial(
        pl.kernel,
        out_type=jax.ShapeDtypeStruct((NC, HALF, D), jnp.float32),
        mesh=_sc_mesh(),
        compiler_params=pltpu.CompilerParams(needs_layout_passes=False),
        scratch_types=[
            pltpu.VMEM((CCAP, B), jnp.int32),
            pltpu.VMEM((CCAP, B), jnp.int32),
            pltpu.VMEM((CCAP, B), jnp.int32),
            pltpu.VMEM((CCAP, B), jnp.int32),
            pltpu.VMEM((1, B), jnp.int32),
            pltpu.VMEM((1, B), jnp.int32),
            pltpu.VMEM((B, D), jnp.float32),
            pltpu.VMEM((B, D), jnp.float32),
            pltpu.VMEM_SHARED((HALF, D), jnp.float32),
            pltpu.SemaphoreType.DMA,
            pltpu.SemaphoreType.DMA,
            pltpu.SemaphoreType.DMA,
            pltpu.SemaphoreType.DMA,
        ],
    )(_edge_body)


def _edge_body(y_hbm, srcl_hbm, dstl_hbm, cnt_hbm, out_hbm,
               sidx0, sidx1, didx0, didx1, cv0, cv1, g0, g1, z_sp,
               sem0, sem1, sems0, sems1):
    c = lax.axis_index("c")
    s = lax.axis_index("s")
    t0 = s * RPH
    pltpu.sync_copy(y_hbm.at[pl.ds(c * HALF + t0, RPH), :],
                    z_sp.at[pl.ds(t0, RPH), :])
    pltpu.sync_copy(srcl_hbm.at[4 * s + c], sidx0)
    pltpu.sync_copy(srcl_hbm.at[4 * s + 2 + c], sidx1)
    pltpu.sync_copy(dstl_hbm.at[4 * s + c], didx0)
    pltpu.sync_copy(dstl_hbm.at[4 * s + 2 + c], didx1)
    pltpu.sync_copy(cnt_hbm.at[2 * s], cv0)
    pltpu.sync_copy(cnt_hbm.at[2 * s + 1], cv1)
    plsc.subcore_barrier()

    for sidx, didx, cvx in ((sidx0, didx0, cv0), (sidx1, didx1, cv1)):
        ct = jnp.max(cvx[0, pl.ds(c * 16, 16)])

        @pl.when(ct >= 2)
        def _(sidx=sidx):
            pltpu.async_copy(y_hbm.at[sidx.at[0]], g0, sem0)
            pltpu.async_copy(y_hbm.at[sidx.at[1]], g1, sem1)

        def pair(jj, carry, sidx=sidx, didx=didx, ct=ct):
            j = jj * 2
            pltpu.make_async_copy(y_hbm.at[sidx.at[0]], g0, sem0).wait()
            pltpu.sync_copy(g0, z_sp.at[didx.at[j]], add=True)

            @pl.when(j + 2 < ct)
            def _():
                pltpu.async_copy(y_hbm.at[sidx.at[j + 2]], g0, sem0)

            pltpu.make_async_copy(y_hbm.at[sidx.at[0]], g1, sem1).wait()
            pltpu.sync_copy(g1, z_sp.at[didx.at[j + 1]], add=True)

            @pl.when(j + 3 < ct)
            def _():
                pltpu.async_copy(y_hbm.at[sidx.at[j + 3]], g1, sem1)

            return carry

        lax.fori_loop(0, ct // 2, pair, 0)

    plsc.subcore_barrier()
    pltpu.sync_copy(z_sp.at[pl.ds(t0, RPH), :],
                    out_hbm.at[c, pl.ds(t0, RPH), :])


_BR = 256
_GRID = NPAD // _BR


def _row_spec():
    return pl.BlockSpec((_BR, D), lambda i: (i, 0))


def _vec_spec():
    return pl.BlockSpec((_BR,), lambda i: (i,))


def _full_spec(shape):
    nd = len(shape)
    return pl.BlockSpec(shape, lambda i: (0,) * nd)


def _mm0_body(x_ref, w_ref, da_ref, db_ref, y_ref):
    dinv = lax.rsqrt(da_ref[...] + db_ref[...] + 1.0)
    xw = lax.dot_general(x_ref[...], w_ref[...], (((1,), (0,)), ((), ())),
                         precision=_HIGH, preferred_element_type=jnp.float32)
    y_ref[...] = xw * dinv[:, None]


def _mm12_body(z_ref, da_ref, db_ref, b_ref, w_ref, y_ref):
    dinv = lax.rsqrt(da_ref[...] + db_ref[...] + 1.0)
    h = jnp.maximum(z_ref[...] * dinv[:, None] + b_ref[...][None, :], 0.0)
    hw = lax.dot_general(h, w_ref[...], (((1,), (0,)), ((), ())),
                         precision=_HIGH, preferred_element_type=jnp.float32)
    y_ref[...] = hw * dinv[:, None]


def _tail_body(z_ref, da_ref, db_ref, b_ref, wn_ref, bn_ref, batch_ref,
               w1_ref, b1_ref, w2_ref, b2_ref,
               np_ref, sums_ref, cnt_ref, fea_ref):
    i = pl.program_id(0)
    dinv = lax.rsqrt(da_ref[...] + db_ref[...] + 1.0)
    h = jnp.maximum(z_ref[...] * dinv[:, None] + b_ref[...][None, :], 0.0)
    np_ref[...] = jnp.sum(h * wn_ref[...][None, :], axis=1) + bn_ref[0]
    onehot = (batch_ref[...][:, None]
              == lax.broadcasted_iota(jnp.int32, (_BR, G), 1)).astype(jnp.float32)
    part = lax.dot_general(onehot, h, (((0,), (0,)), ((), ())),
                           precision=_HIGH, preferred_element_type=jnp.float32)
    cpart = jnp.sum(onehot, axis=0)

    @pl.when(i == 0)
    def _():
        sums_ref[...] = part
        cnt_ref[...] = cpart

    @pl.when(i > 0)
    def _():
        sums_ref[...] += part
        cnt_ref[...] += cpart

    @pl.when(i == _GRID - 1)
    def _():
        mean = sums_ref[...] / jnp.maximum(cnt_ref[...], 1.0)[:, None]
        t = lax.dot_general(mean, w1_ref[...], (((1,), (0,)), ((), ())),
                            precision=_HIGH, preferred_element_type=jnp.float32)
        t = jnp.maximum(t + b1_ref[...][None, :], 0.0)
        fea_ref[...] = lax.dot_general(t, w2_ref[...], (((1,), (0,)), ((), ())),
                                       precision=_HIGH,
                                       preferred_element_type=jnp.float32) \
            + b2_ref[...][None, :]


def _mm0(x_pad, w, dega, degb):
    return pl.pallas_call(
        _mm0_body,
        grid=(_GRID,),
        in_specs=[_row_spec(), _full_spec((D, D)), _vec_spec(), _vec_spec()],
        out_specs=_row_spec(),
        out_shape=jax.ShapeDtypeStruct((NPAD, D), jnp.float32),
    )(x_pad, w, dega, degb)


def _mm12(z, dega, degb, b, w):
    return pl.pallas_call(
        _mm12_body,
        grid=(_GRID,),
        in_specs=[_row_spec(), _vec_spec(), _vec_spec(),
                  _full_spec((D,)), _full_spec((D, D))],
        out_specs=_row_spec(),
        out_shape=jax.ShapeDtypeStruct((NPAD, D), jnp.float32),
    )(z, dega, degb, b, w)


def _tail(z, dega, degb, b, wn, bn, batch_pad, w1, b1, w2, b2):
    return pl.pallas_call(
        _tail_body,
        grid=(_GRID,),
        in_specs=[_row_spec(), _vec_spec(), _vec_spec(), _full_spec((D,)),
                  _full_spec((D,)), _full_spec((8,)), _vec_spec(),
                  _full_spec((D, 256)), _full_spec((256,)),
                  _full_spec((256, D)), _full_spec((D,))],
        out_specs=[_vec_spec(), _full_spec((G, D)), _full_spec((G,)),
                   _full_spec((G, D))],
        out_shape=[jax.ShapeDtypeStruct((NPAD,), jnp.float32),
                   jax.ShapeDtypeStruct((G, D), jnp.float32),
                   jax.ShapeDtypeStruct((G,), jnp.float32),
                   jax.ShapeDtypeStruct((G, D), jnp.float32)],
    )(z, dega, degb, b, wn, bn, batch_pad, w1, b1, w2, b2)


def kernel(x, edge_index, batch, W0, b0, W1, b1, W2, b2,
           W_f1, b_f1, W_f2, b_f2, W_n, b_n):
    zpad = jnp.zeros((PADB, D), jnp.float32)
    x_adj = jnp.concatenate([x[:NREAL_H], zpad, x[NREAL_H:], zpad])
    gfill = jnp.full((PADB,), G, jnp.int32)
    batch_adj = jnp.concatenate([batch[:NREAL_H], gfill, batch[NREAL_H:], gfill])
    src = edge_index[0]
    dst = edge_index[1]
    src_adj = jnp.where(src >= NREAL_H, src + PADB, src)
    dst_adj = jnp.where(dst >= NREAL_H, dst + PADB, dst)
    pad_i = jnp.arange(EPAD - E, dtype=jnp.int32)
    pad_src = (pad_i * 97) % NREAL_H
    pad_dst = NREAL_H + pad_i % PADB
    srcw = jnp.concatenate([src_adj, pad_src]).reshape(NW, NCHUNK, B)
    dstw = jnp.concatenate([dst_adj, pad_dst]).reshape(NW, NCHUNK, B)

    bn8 = jnp.pad(b_n, (0, 7))
    wn = W_n[:, 0]

    deg2, srcl, dstl, cnts = _get_part_kernel()(srcw, dstw)
    dega, degb = deg2[0], deg2[1]
    edge_k = _get_edge_kernel()

    y0 = _mm0(x_adj, W0, dega, degb)
    z0 = edge_k(y0, srcl, dstl, cnts).reshape(NPAD, D)
    y1 = _mm12(z0, dega, degb, b0, W1)
    z1 = edge_k(y1, srcl, dstl, cnts).reshape(NPAD, D)
    y2 = _mm12(z1, dega, degb, b1, W2)
    z2 = edge_k(y2, srcl, dstl, cnts).reshape(NPAD, D)

    node_prob, _, _, fea = _tail(z2, dega, degb, b2, wn, bn8, batch_adj,
                                 W_f1, b_f1, W_f2, b_f2)
    node_prob = jnp.concatenate([node_prob[:NREAL_H],
                                 node_prob[HALF:HALF + NREAL_H]])
    return (node_prob, fea)


# speedup vs baseline: 1.3419x; 1.0571x over previous
import functools

import jax
import jax.numpy as jnp
from jax import lax
from jax.experimental import pallas as pl
from jax.experimental.pallas import tpu as pltpu
from jax.experimental.pallas import tpu_sc as plsc

N = 10000
D = 128
G = 64
NC = 2
NS = 16
NW = NC * NS
B = 128
HALF = 5120
NPAD = 2 * HALF
PADR = NPAD - N
RPH = HALF // NS
RPT = NPAD // NS
E = 320000
NCHUNK = -(-E // (NW * B))
NCHUNK += NCHUNK % 2
EPAD = NW * B * NCHUNK
CCAP = NCHUNK + 2

_HIGH = lax.Precision.HIGHEST


def _sc_mesh():
    return plsc.VectorSubcoreMesh(core_axis_name="c", subcore_axis_name="s",
                                  num_cores=NC, num_subcores=NS)


@functools.cache
def _get_part_kernel():
    return functools.partial(
        pl.kernel,
        out_type=(
            jax.ShapeDtypeStruct((NC, NPAD), jnp.float32),
            jax.ShapeDtypeStruct((2 * NW, CCAP, B), jnp.int32),
            jax.ShapeDtypeStruct((2 * NW, CCAP, B), jnp.int32),
            jax.ShapeDtypeStruct((NW, 1, B), jnp.int32),
        ),
        mesh=_sc_mesh(),
        compiler_params=pltpu.CompilerParams(needs_layout_passes=False),
        scratch_types=[
            pltpu.VMEM((NCHUNK, B), jnp.int32),
            pltpu.VMEM((NCHUNK, B), jnp.int32),
            pltpu.VMEM((CCAP, B), jnp.int32),
            pltpu.VMEM((CCAP, B), jnp.int32),
            pltpu.VMEM((CCAP, B), jnp.int32),
            pltpu.VMEM((CCAP, B), jnp.int32),
            pltpu.VMEM((B,), jnp.float32),
            pltpu.VMEM((RPT,), jnp.float32),
            pltpu.VMEM((1, B), jnp.int32),
            pltpu.VMEM_SHARED((NPAD,), jnp.float32),
        ],
    )(_part_body)


def _part_body(srcw_hbm, dstw_hbm, deg_hbm, srcl_hbm, dstl_hbm, cnt_hbm,
               src_v, dst_v, s0_v, d0_v, s1_v, d1_v, ones_v, zero_v, cnt_v,
               deg_sp):
    c = lax.axis_index("c")
    s = lax.axis_index("s")
    wid = c * NS + s
    t0 = s * RPT
    for k in range(B // 16):
        ones_v[pl.ds(k * 16, 16)] = jnp.ones((16,), jnp.float32)
    for k in range(RPT // 16):
        zero_v[pl.ds(k * 16, 16)] = jnp.zeros((16,), jnp.float32)
    pltpu.sync_copy(srcw_hbm.at[wid], src_v)
    pltpu.sync_copy(dstw_hbm.at[wid], dst_v)
    pltpu.sync_copy(zero_v, deg_sp.at[pl.ds(t0, RPT)])
    plsc.subcore_barrier()

    iota16 = lax.iota(jnp.int32, 16)

    def scat2(ref_s, ref_d, pos, vs, vd, m=None):
        pj = lax.shift_right_logical(pos, 7)
        pk = lax.bitwise_and(pos, 127)
        if m is None:
            plsc.store_scatter(ref_s, [pj, pk], vs)
            plsc.store_scatter(ref_d, [pj, pk], vd)
        else:
            plsc.store_scatter(ref_s, [pj, pk], vs, mask=m)
            plsc.store_scatter(ref_d, [pj, pk], vd, mask=m)

    def chunk(j, offs):
        off0, off1 = offs
        pltpu.sync_copy(ones_v, deg_sp.at[dst_v.at[j]], add=True)
        for k in range(B // 16):
            vs = src_v[j, pl.ds(k * 16, 16)]
            vd = dst_v[j, pl.ds(k * 16, 16)]
            m0 = vd < HALF
            cs0 = plsc.cumsum(m0.astype(jnp.int32))
            scat2(s0_v, d0_v, off0 + cs0 - 1, vs, vd, m0)
            off0 = off0 + jnp.max(cs0)
            m1 = jnp.logical_not(m0)
            cs1 = plsc.cumsum(m1.astype(jnp.int32))
            scat2(s1_v, d1_v, off1 + cs1 - 1, vs, vd - HALF, m1)
            off1 = off1 + jnp.max(cs1)
        return off0, off1

    off0, off1 = lax.fori_loop(0, NCHUNK, chunk, (jnp.int32(0), jnp.int32(0)))

    for k in range(2 * B // 16):
        ps = N + (iota16 * 7 + k * 13) % PADR
        pd = (iota16 * 977 + k * 131) % HALF
        scat2(s0_v, d0_v, off0 + k * 16 + iota16, ps, pd)
        scat2(s1_v, d1_v, off1 + k * 16 + iota16, ps, pd)
    ch0 = (off0 + B - 1) // B
    ch0 = ch0 + (ch0 & 1)
    ch1 = (off1 + B - 1) // B
    ch1 = ch1 + (ch1 & 1)
    cnt_v[0, pl.ds(0, 16)] = jnp.full((16,), ch0, jnp.int32)
    cnt_v[0, pl.ds(16, 16)] = jnp.full((16,), ch1, jnp.int32)

    pltpu.sync_copy(s0_v, srcl_hbm.at[2 * wid])
    pltpu.sync_copy(d0_v, dstl_hbm.at[2 * wid])
    pltpu.sync_copy(s1_v, srcl_hbm.at[2 * wid + 1])
    pltpu.sync_copy(d1_v, dstl_hbm.at[2 * wid + 1])
    pltpu.sync_copy(cnt_v, cnt_hbm.at[wid])
    plsc.subcore_barrier()
    pltpu.sync_copy(deg_sp.at[pl.ds(t0, RPT)], deg_hbm.at[c, pl.ds(t0, RPT)])


@functools.cache
def _get_edge_kernel():
    return functools.partial(
        pl.kernel,
        out_type=jax.ShapeDtypeStruct((NC, HALF, D), jnp.float32),
        mesh=_sc_mesh(),
        compiler_params=pltpu.CompilerParams(needs_layout_passes=False),
        scratch_types=[
            pltpu.VMEM((CCAP, B), jnp.int32),
            pltpu.VMEM((CCAP, B), jnp.int32),
            pltpu.VMEM((CCAP, B), jnp.int32),
            pltpu.VMEM((CCAP, B), jnp.int32),
            pltpu.VMEM((1, B), jnp.int32),
            pltpu.VMEM((1, B), jnp.int32),
            pltpu.VMEM((B, D), jnp.float32),
            pltpu.VMEM((B, D), jnp.float32),
            pltpu.VMEM_SHARED((HALF, D), jnp.float32),
            pltpu.SemaphoreType.DMA,
            pltpu.SemaphoreType.DMA,
            pltpu.SemaphoreType.DMA,
            pltpu.SemaphoreType.DMA,
        ],
    )(_edge_body)


def _edge_body(y_hbm, srcl_hbm, dstl_hbm, cnt_hbm, out_hbm,
               sidx0, sidx1, didx0, didx1, cv0, cv1, g0, g1, z_sp,
               sem0, sem1, sems0, sems1):
    c = lax.axis_index("c")
    s = lax.axis_index("s")
    t0 = s * RPH
    pltpu.sync_copy(y_hbm.at[pl.ds(c * HALF + t0, RPH), :],
                    z_sp.at[pl.ds(t0, RPH), :])
    pltpu.sync_copy(srcl_hbm.at[4 * s + c], sidx0)
    pltpu.sync_copy(srcl_hbm.at[4 * s + 2 + c], sidx1)
    pltpu.sync_copy(dstl_hbm.at[4 * s + c], didx0)
    pltpu.sync_copy(dstl_hbm.at[4 * s + 2 + c], didx1)
    pltpu.sync_copy(cnt_hbm.at[2 * s], cv0)
    pltpu.sync_copy(cnt_hbm.at[2 * s + 1], cv1)
    plsc.subcore_barrier()

    for sidx, didx, cvx in ((sidx0, didx0, cv0), (sidx1, didx1, cv1)):
        ct = jnp.max(cvx[0, pl.ds(c * 16, 16)])

        @pl.when(ct >= 2)
        def _(sidx=sidx):
            pltpu.async_copy(y_hbm.at[sidx.at[0]], g0, sem0)
            pltpu.async_copy(y_hbm.at[sidx.at[1]], g1, sem1)

        def pair(jj, carry, sidx=sidx, didx=didx, ct=ct):
            j = jj * 2
            pltpu.make_async_copy(y_hbm.at[sidx.at[0]], g0, sem0).wait()
            pltpu.sync_copy(g0, z_sp.at[didx.at[j]], add=True)

            @pl.when(j + 2 < ct)
            def _():
                pltpu.async_copy(y_hbm.at[sidx.at[j + 2]], g0, sem0)

            pltpu.make_async_copy(y_hbm.at[sidx.at[0]], g1, sem1).wait()
            pltpu.sync_copy(g1, z_sp.at[didx.at[j + 1]], add=True)

            @pl.when(j + 3 < ct)
            def _():
                pltpu.async_copy(y_hbm.at[sidx.at[j + 3]], g1, sem1)

            return carry

        lax.fori_loop(0, ct // 2, pair, 0)

    plsc.subcore_barrier()
    pltpu.sync_copy(z_sp.at[pl.ds(t0, RPH), :],
                    out_hbm.at[c, pl.ds(t0, RPH), :])


_BR = 256
_GRID = NPAD // _BR


def _row_spec():
    return pl.BlockSpec((_BR, D), lambda i: (i, 0))


def _vec_spec():
    return pl.BlockSpec((_BR,), lambda i: (i,))


def _full_spec(shape):
    nd = len(shape)
    return pl.BlockSpec(shape, lambda i: (0,) * nd)


def _row_mask():
    rid = pl.program_id(0) * _BR + lax.broadcasted_iota(jnp.int32, (_BR,), 0)
    return (rid < N).astype(jnp.float32)


def _mm0_body(x_ref, w_ref, da_ref, db_ref, y_ref):
    dinv = lax.rsqrt(da_ref[...] + db_ref[...] + 1.0) * _row_mask()
    xw = lax.dot_general(x_ref[...], w_ref[...], (((1,), (0,)), ((), ())),
                         precision=_HIGH, preferred_element_type=jnp.float32)
    y_ref[...] = xw * dinv[:, None]


def _mm12_body(z_ref, da_ref, db_ref, b_ref, w_ref, y_ref):
    dinv = lax.rsqrt(da_ref[...] + db_ref[...] + 1.0)
    h = jnp.maximum(z_ref[...] * dinv[:, None] + b_ref[...][None, :], 0.0)
    hw = lax.dot_general(h, w_ref[...], (((1,), (0,)), ((), ())),
                         precision=_HIGH, preferred_element_type=jnp.float32)
    y_ref[...] = hw * (dinv * _row_mask())[:, None]


def _tail_body(z_ref, da_ref, db_ref, b_ref, wn_ref, bn_ref, batch_ref,
               w1_ref, b1_ref, w2_ref, b2_ref,
               np_ref, sums_ref, cnt_ref, fea_ref):
    i = pl.program_id(0)
    dinv = lax.rsqrt(da_ref[...] + db_ref[...] + 1.0)
    h = jnp.maximum(z_ref[...] * dinv[:, None] + b_ref[...][None, :], 0.0)
    np_ref[...] = jnp.sum(h * wn_ref[...][None, :], axis=1) + bn_ref[0]
    onehot = (batch_ref[...][:, None]
              == lax.broadcasted_iota(jnp.int32, (_BR, G), 1)).astype(jnp.float32)
    part = lax.dot_general(onehot, h, (((0,), (0,)), ((), ())),
                           precision=_HIGH, preferred_element_type=jnp.float32)
    cpart = jnp.sum(onehot, axis=0)

    @pl.when(i == 0)
    def _():
        sums_ref[...] = part
        cnt_ref[...] = cpart

    @pl.when(i > 0)
    def _():
        sums_ref[...] += part
        cnt_ref[...] += cpart

    @pl.when(i == _GRID - 1)
    def _():
        mean = sums_ref[...] / jnp.maximum(cnt_ref[...], 1.0)[:, None]
        t = lax.dot_general(mean, w1_ref[...], (((1,), (0,)), ((), ())),
                            precision=_HIGH, preferred_element_type=jnp.float32)
        t = jnp.maximum(t + b1_ref[...][None, :], 0.0)
        fea_ref[...] = lax.dot_general(t, w2_ref[...], (((1,), (0,)), ((), ())),
                                       precision=_HIGH,
                                       preferred_element_type=jnp.float32) \
            + b2_ref[...][None, :]


def _mm0(x_pad, w, dega, degb):
    return pl.pallas_call(
        _mm0_body,
        grid=(_GRID,),
        in_specs=[_row_spec(), _full_spec((D, D)), _vec_spec(), _vec_spec()],
        out_specs=_row_spec(),
        out_shape=jax.ShapeDtypeStruct((NPAD, D), jnp.float32),
    )(x_pad, w, dega, degb)


def _mm12(z, dega, degb, b, w):
    return pl.pallas_call(
        _mm12_body,
        grid=(_GRID,),
        in_specs=[_row_spec(), _vec_spec(), _vec_spec(),
                  _full_spec((D,)), _full_spec((D, D))],
        out_specs=_row_spec(),
        out_shape=jax.ShapeDtypeStruct((NPAD, D), jnp.float32),
    )(z, dega, degb, b, w)


def _tail(z, dega, degb, b, wn, bn, batch_pad, w1, b1, w2, b2):
    return pl.pallas_call(
        _tail_body,
        grid=(_GRID,),
        in_specs=[_row_spec(), _vec_spec(), _vec_spec(), _full_spec((D,)),
                  _full_spec((D,)), _full_spec((8,)), _vec_spec(),
                  _full_spec((D, 256)), _full_spec((256,)),
                  _full_spec((256, D)), _full_spec((D,))],
        out_specs=[_vec_spec(), _full_spec((G, D)), _full_spec((G,)),
                   _full_spec((G, D))],
        out_shape=[jax.ShapeDtypeStruct((NPAD,), jnp.float32),
                   jax.ShapeDtypeStruct((G, D), jnp.float32),
                   jax.ShapeDtypeStruct((G,), jnp.float32),
                   jax.ShapeDtypeStruct((G, D), jnp.float32)],
    )(z, dega, degb, b, wn, bn, batch_pad, w1, b1, w2, b2)


def kernel(x, edge_index, batch, W0, b0, W1, b1, W2, b2,
           W_f1, b_f1, W_f2, b_f2, W_n, b_n):
    x_adj = jnp.pad(x, ((0, PADR), (0, 0)))
    batch_adj = jnp.pad(batch, (0, PADR), constant_values=G)
    pad_i = jnp.arange(EPAD - E, dtype=jnp.int32)
    pad_src = N + pad_i % PADR
    pad_dst = N + (pad_i * 7) % PADR
    srcw = jnp.concatenate([edge_index[0], pad_src]).reshape(NW, NCHUNK, B)
    dstw = jnp.concatenate([edge_index[1], pad_dst]).reshape(NW, NCHUNK, B)

    bn8 = jnp.pad(b_n, (0, 7))
    wn = W_n[:, 0]

    deg2, srcl, dstl, cnts = _get_part_kernel()(srcw, dstw)
    dega, degb = deg2[0], deg2[1]
    edge_k = _get_edge_kernel()

    y0 = _mm0(x_adj, W0, dega, degb)
    z0 = edge_k(y0, srcl, dstl, cnts).reshape(NPAD, D)
    y1 = _mm12(z0, dega, degb, b0, W1)
    z1 = edge_k(y1, srcl, dstl, cnts).reshape(NPAD, D)
    y2 = _mm12(z1, dega, degb, b1, W2)
    z2 = edge_k(y2, srcl, dstl, cnts).reshape(NPAD, D)

    node_prob, _, _, fea = _tail(z2, dega, degb, b2, wn, bn8, batch_adj,
                                 W_f1, b_f1, W_f2, b_f2)
    return (node_prob[:N], fea)


# speedup vs baseline: 1.3582x; 1.0122x over previous
import functools

import jax
import jax.numpy as jnp
from jax import lax
from jax.experimental import pallas as pl
from jax.experimental.pallas import tpu as pltpu
from jax.experimental.pallas import tpu_sc as plsc

N = 10000
D = 128
G = 64
NC = 2
NS = 16
NW = NC * NS
B = 128
HALF = 5120
NPAD = 2 * HALF
PADR = NPAD - N
RPH = HALF // NS
RPT = NPAD // NS
E = 320000
NCHUNK = -(-E // (NW * B))
NCHUNK += NCHUNK % 2
EPAD = NW * B * NCHUNK
CCAP = NCHUNK + 2

_DEFP = None


def _rsqrt(x):
    r = lax.rsqrt(x)
    return r * (1.5 - 0.5 * x * r * r)


def _sc_mesh():
    return plsc.VectorSubcoreMesh(core_axis_name="c", subcore_axis_name="s",
                                  num_cores=NC, num_subcores=NS)


@functools.cache
def _get_part_kernel():
    return functools.partial(
        pl.kernel,
        out_type=(
            jax.ShapeDtypeStruct((NC, NPAD), jnp.float32),
            jax.ShapeDtypeStruct((2 * NW, CCAP, B), jnp.int32),
            jax.ShapeDtypeStruct((2 * NW, CCAP, B), jnp.int32),
            jax.ShapeDtypeStruct((NW, 1, B), jnp.int32),
        ),
        mesh=_sc_mesh(),
        compiler_params=pltpu.CompilerParams(needs_layout_passes=False),
        scratch_types=[
            pltpu.VMEM((NCHUNK, B), jnp.int32),
            pltpu.VMEM((NCHUNK, B), jnp.int32),
            pltpu.VMEM((CCAP, B), jnp.int32),
            pltpu.VMEM((CCAP, B), jnp.int32),
            pltpu.VMEM((CCAP, B), jnp.int32),
            pltpu.VMEM((CCAP, B), jnp.int32),
            pltpu.VMEM((B,), jnp.float32),
            pltpu.VMEM((RPT,), jnp.float32),
            pltpu.VMEM((1, B), jnp.int32),
            pltpu.VMEM_SHARED((NPAD,), jnp.float32),
        ],
    )(_part_body)


def _part_body(srcw_hbm, dstw_hbm, deg_hbm, srcl_hbm, dstl_hbm, cnt_hbm,
               src_v, dst_v, s0_v, d0_v, s1_v, d1_v, ones_v, zero_v, cnt_v,
               deg_sp):
    c = lax.axis_index("c")
    s = lax.axis_index("s")
    wid = c * NS + s
    t0 = s * RPT
    for k in range(B // 16):
        ones_v[pl.ds(k * 16, 16)] = jnp.ones((16,), jnp.float32)
    for k in range(RPT // 16):
        zero_v[pl.ds(k * 16, 16)] = jnp.zeros((16,), jnp.float32)
    pltpu.sync_copy(srcw_hbm.at[wid], src_v)
    pltpu.sync_copy(dstw_hbm.at[wid], dst_v)
    pltpu.sync_copy(zero_v, deg_sp.at[pl.ds(t0, RPT)])
    plsc.subcore_barrier()

    iota16 = lax.iota(jnp.int32, 16)

    def scat2(ref_s, ref_d, pos, vs, vd, m=None):
        pj = lax.shift_right_logical(pos, 7)
        pk = lax.bitwise_and(pos, 127)
        if m is None:
            plsc.store_scatter(ref_s, [pj, pk], vs)
            plsc.store_scatter(ref_d, [pj, pk], vd)
        else:
            plsc.store_scatter(ref_s, [pj, pk], vs, mask=m)
            plsc.store_scatter(ref_d, [pj, pk], vd, mask=m)

    def chunk(j, offs):
        off0, off1 = offs
        pltpu.sync_copy(ones_v, deg_sp.at[dst_v.at[j]], add=True)
        for k in range(B // 16):
            vs = src_v[j, pl.ds(k * 16, 16)]
            vd = dst_v[j, pl.ds(k * 16, 16)]
            m0 = vd < HALF
            cs0 = plsc.cumsum(m0.astype(jnp.int32))
            scat2(s0_v, d0_v, off0 + cs0 - 1, vs, vd, m0)
            off0 = off0 + jnp.max(cs0)
            m1 = jnp.logical_not(m0)
            cs1 = plsc.cumsum(m1.astype(jnp.int32))
            scat2(s1_v, d1_v, off1 + cs1 - 1, vs, vd - HALF, m1)
            off1 = off1 + jnp.max(cs1)
        return off0, off1

    off0, off1 = lax.fori_loop(0, NCHUNK, chunk, (jnp.int32(0), jnp.int32(0)))

    for k in range(2 * B // 16):
        ps = N + (iota16 * 7 + k * 13) % PADR
        pd = (iota16 * 977 + k * 131) % HALF
        scat2(s0_v, d0_v, off0 + k * 16 + iota16, ps, pd)
        scat2(s1_v, d1_v, off1 + k * 16 + iota16, ps, pd)
    ch0 = (off0 + B - 1) // B
    ch0 = ch0 + (ch0 & 1)
    ch1 = (off1 + B - 1) // B
    ch1 = ch1 + (ch1 & 1)
    cnt_v[0, pl.ds(0, 16)] = jnp.full((16,), ch0, jnp.int32)
    cnt_v[0, pl.ds(16, 16)] = jnp.full((16,), ch1, jnp.int32)

    pltpu.sync_copy(s0_v, srcl_hbm.at[2 * wid])
    pltpu.sync_copy(d0_v, dstl_hbm.at[2 * wid])
    pltpu.sync_copy(s1_v, srcl_hbm.at[2 * wid + 1])
    pltpu.sync_copy(d1_v, dstl_hbm.at[2 * wid + 1])
    pltpu.sync_copy(cnt_v, cnt_hbm.at[wid])
    plsc.subcore_barrier()
    pltpu.sync_copy(deg_sp.at[pl.ds(t0, RPT)], deg_hbm.at[c, pl.ds(t0, RPT)])


@functools.cache
def _get_edge_kernel():
    return functools.partial(
        pl.kernel,
        out_type=jax.ShapeDtypeStruct((NC, HALF, D), jnp.float32),
        mesh=_sc_mesh(),
        compiler_params=pltpu.CompilerParams(needs_layout_passes=False),
        scratch_types=[
            pltpu.VMEM((CCAP, B), jnp.int32),
            pltpu.VMEM((CCAP, B), jnp.int32),
            pltpu.VMEM((CCAP, B), jnp.int32),
            pltpu.VMEM((CCAP, B), jnp.int32),
            pltpu.VMEM((1, B), jnp.int32),
            pltpu.VMEM((1, B), jnp.int32),
            pltpu.VMEM((B, D), jnp.float32),
            pltpu.VMEM((B, D), jnp.float32),
            pltpu.VMEM_SHARED((HALF, D), jnp.float32),
            pltpu.SemaphoreType.DMA,
            pltpu.SemaphoreType.DMA,
            pltpu.SemaphoreType.DMA,
            pltpu.SemaphoreType.DMA,
        ],
    )(_edge_body)


def _edge_body(y_hbm, srcl_hbm, dstl_hbm, cnt_hbm, out_hbm,
               sidx0, sidx1, didx0, didx1, cv0, cv1, g0, g1, z_sp,
               sem0, sem1, sems0, sems1):
    c = lax.axis_index("c")
    s = lax.axis_index("s")
    t0 = s * RPH
    pltpu.sync_copy(y_hbm.at[pl.ds(c * HALF + t0, RPH), :],
                    z_sp.at[pl.ds(t0, RPH), :])
    pltpu.sync_copy(srcl_hbm.at[4 * s + c], sidx0)
    pltpu.sync_copy(srcl_hbm.at[4 * s + 2 + c], sidx1)
    pltpu.sync_copy(dstl_hbm.at[4 * s + c], didx0)
    pltpu.sync_copy(dstl_hbm.at[4 * s + 2 + c], didx1)
    pltpu.sync_copy(cnt_hbm.at[2 * s], cv0)
    pltpu.sync_copy(cnt_hbm.at[2 * s + 1], cv1)
    plsc.subcore_barrier()

    for sidx, didx, cvx in ((sidx0, didx0, cv0), (sidx1, didx1, cv1)):
        ct = jnp.max(cvx[0, pl.ds(c * 16, 16)])

        @pl.when(ct >= 2)
        def _(sidx=sidx):
            pltpu.async_copy(y_hbm.at[sidx.at[0]], g0, sem0)
            pltpu.async_copy(y_hbm.at[sidx.at[1]], g1, sem1)

        def pair(jj, carry, sidx=sidx, didx=didx, ct=ct):
            j = jj * 2
            pltpu.make_async_copy(y_hbm.at[sidx.at[0]], g0, sem0).wait()
            pltpu.sync_copy(g0, z_sp.at[didx.at[j]], add=True)

            @pl.when(j + 2 < ct)
            def _():
                pltpu.async_copy(y_hbm.at[sidx.at[j + 2]], g0, sem0)

            pltpu.make_async_copy(y_hbm.at[sidx.at[0]], g1, sem1).wait()
            pltpu.sync_copy(g1, z_sp.at[didx.at[j + 1]], add=True)

            @pl.when(j + 3 < ct)
            def _():
                pltpu.async_copy(y_hbm.at[sidx.at[j + 3]], g1, sem1)

            return carry

        lax.fori_loop(0, ct // 2, pair, 0)

    plsc.subcore_barrier()
    pltpu.sync_copy(z_sp.at[pl.ds(t0, RPH), :],
                    out_hbm.at[c, pl.ds(t0, RPH), :])


_BR = 256
_GRID = NPAD // _BR


def _row_spec():
    return pl.BlockSpec((_BR, D), lambda i: (i, 0))


def _vec_spec():
    return pl.BlockSpec((_BR,), lambda i: (i,))


def _full_spec(shape):
    nd = len(shape)
    return pl.BlockSpec(shape, lambda i: (0,) * nd)


def _row_mask():
    rid = pl.program_id(0) * _BR + lax.broadcasted_iota(jnp.int32, (_BR,), 0)
    return (rid < N).astype(jnp.float32)


def _mm0_body(x_ref, w_ref, da_ref, db_ref, y_ref):
    dinv = _rsqrt(da_ref[...] + db_ref[...] + 1.0) * _row_mask()
    xw = lax.dot_general(x_ref[...], w_ref[...], (((1,), (0,)), ((), ())),
                         precision=_DEFP, preferred_element_type=jnp.float32)
    y_ref[...] = xw * dinv[:, None]


def _mm12_body(z_ref, da_ref, db_ref, b_ref, w_ref, y_ref):
    dinv = _rsqrt(da_ref[...] + db_ref[...] + 1.0)
    h = jnp.maximum(z_ref[...] * dinv[:, None] + b_ref[...][None, :], 0.0)
    hw = lax.dot_general(h, w_ref[...], (((1,), (0,)), ((), ())),
                         precision=_DEFP, preferred_element_type=jnp.float32)
    y_ref[...] = hw * (dinv * _row_mask())[:, None]


def _tail_body(z_ref, da_ref, db_ref, b_ref, wn_ref, bn_ref, batch_ref,
               w1_ref, b1_ref, w2_ref, b2_ref,
               np_ref, sums_ref, cnt_ref, fea_ref):
    i = pl.program_id(0)
    dinv = _rsqrt(da_ref[...] + db_ref[...] + 1.0)
    h = jnp.maximum(z_ref[...] * dinv[:, None] + b_ref[...][None, :], 0.0)
    np_ref[...] = jnp.sum(h * wn_ref[...][None, :], axis=1) + bn_ref[0]
    onehot = (batch_ref[...][:, None]
              == lax.broadcasted_iota(jnp.int32, (_BR, G), 1)).astype(jnp.float32)
    part = lax.dot_general(onehot, h, (((0,), (0,)), ((), ())),
                           precision=_DEFP, preferred_element_type=jnp.float32)
    cpart = jnp.sum(onehot, axis=0)

    @pl.when(i == 0)
    def _():
        sums_ref[...] = part
        cnt_ref[...] = cpart

    @pl.when(i > 0)
    def _():
        sums_ref[...] += part
        cnt_ref[...] += cpart

    @pl.when(i == _GRID - 1)
    def _():
        mean = sums_ref[...] / jnp.maximum(cnt_ref[...], 1.0)[:, None]
        t = lax.dot_general(mean, w1_ref[...], (((1,), (0,)), ((), ())),
                            precision=_DEFP, preferred_element_type=jnp.float32)
        t = jnp.maximum(t + b1_ref[...][None, :], 0.0)
        fea_ref[...] = lax.dot_general(t, w2_ref[...], (((1,), (0,)), ((), ())),
                                       precision=_DEFP,
                                       preferred_element_type=jnp.float32) \
            + b2_ref[...][None, :]


def _mm0(x_pad, w, dega, degb):
    return pl.pallas_call(
        _mm0_body,
        grid=(_GRID,),
        in_specs=[_row_spec(), _full_spec((D, D)), _vec_spec(), _vec_spec()],
        out_specs=_row_spec(),
        out_shape=jax.ShapeDtypeStruct((NPAD, D), jnp.float32),
    )(x_pad, w, dega, degb)


def _mm12(z, dega, degb, b, w):
    return pl.pallas_call(
        _mm12_body,
        grid=(_GRID,),
        in_specs=[_row_spec(), _vec_spec(), _vec_spec(),
                  _full_spec((D,)), _full_spec((D, D))],
        out_specs=_row_spec(),
        out_shape=jax.ShapeDtypeStruct((NPAD, D), jnp.float32),
    )(z, dega, degb, b, w)


def _tail(z, dega, degb, b, wn, bn, batch_pad, w1, b1, w2, b2):
    return pl.pallas_call(
        _tail_body,
        grid=(_GRID,),
        in_specs=[_row_spec(), _vec_spec(), _vec_spec(), _full_spec((D,)),
                  _full_spec((D,)), _full_spec((8,)), _vec_spec(),
                  _full_spec((D, 256)), _full_spec((256,)),
                  _full_spec((256, D)), _full_spec((D,))],
        out_specs=[_vec_spec(), _full_spec((G, D)), _full_spec((G,)),
                   _full_spec((G, D))],
        out_shape=[jax.ShapeDtypeStruct((NPAD,), jnp.float32),
                   jax.ShapeDtypeStruct((G, D), jnp.float32),
                   jax.ShapeDtypeStruct((G,), jnp.float32),
                   jax.ShapeDtypeStruct((G, D), jnp.float32)],
    )(z, dega, degb, b, wn, bn, batch_pad, w1, b1, w2, b2)


def kernel(x, edge_index, batch, W0, b0, W1, b1, W2, b2,
           W_f1, b_f1, W_f2, b_f2, W_n, b_n):
    x_adj = jnp.pad(x, ((0, PADR), (0, 0)))
    batch_adj = jnp.pad(batch, (0, PADR), constant_values=G)
    pad_i = jnp.arange(EPAD - E, dtype=jnp.int32)
    pad_src = N + pad_i % PADR
    pad_dst = N + (pad_i * 7) % PADR
    srcw = jnp.concatenate([edge_index[0], pad_src]).reshape(NW, NCHUNK, B)
    dstw = jnp.concatenate([edge_index[1], pad_dst]).reshape(NW, NCHUNK, B)

    bn8 = jnp.pad(b_n, (0, 7))
    wn = W_n[:, 0]

    deg2, srcl, dstl, cnts = _get_part_kernel()(srcw, dstw)
    dega, degb = deg2[0], deg2[1]
    edge_k = _get_edge_kernel()

    y0 = _mm0(x_adj, W0, dega, degb)
    z0 = edge_k(y0, srcl, dstl, cnts).reshape(NPAD, D)
    y1 = _mm12(z0, dega, degb, b0, W1)
    z1 = edge_k(y1, srcl, dstl, cnts).reshape(NPAD, D)
    y2 = _mm12(z1, dega, degb, b1, W2)
    z2 = edge_k(y2, srcl, dstl, cnts).reshape(NPAD, D)

    node_prob, _, _, fea = _tail(z2, dega, degb, b2, wn, bn8, batch_adj,
                                 W_f1, b_f1, W_f2, b_f2)
    return (node_prob[:N], fea)
